# Initial kernel scaffold; baseline (speedup 1.0000x reference)
import jax
import jax.numpy as jnp
from jax.experimental import pallas as pl


def kernel(x, edge_index, fc_W, fc_b, W1, b1, W2, b2):
    raise NotImplementedError("write your pallas kernel here")


# speedup vs baseline: 16.8941x; 16.8941x over previous
import functools

import jax
import jax.numpy as jnp
from jax import lax
from jax.experimental import pallas as pl
from jax.experimental.pallas import tpu as pltpu
from jax.experimental.pallas import tpu_sc as plsc

NC = 2
NS = 16
NW = NC * NS
EB = 128
D = 128
N = 10000
RPT = 640
NPAD = NS * RPT
R = 1000


def _vsmesh():
    return plsc.VectorSubcoreMesh(core_axis_name="c", subcore_axis_name="s")


def _sc_degree(col2d):
    er = col2d.shape[0]
    full, rem = divmod(er, NW)

    @functools.partial(
        pl.kernel,
        mesh=_vsmesh(),
        out_type=jax.ShapeDtypeStruct((NC * NPAD,), jnp.float32),
        scratch_types=[
            pltpu.VMEM((EB,), jnp.int32),
            pltpu.VMEM((EB,), jnp.float32),
            pltpu.VMEM((RPT,), jnp.float32),
            pltpu.VMEM_SHARED((NPAD,), jnp.float32),
        ],
    )
    def deg_kernel(col_hbm, out_hbm, colv, onesv, stg, deg_sh):
        c = lax.axis_index("c")
        s = lax.axis_index("s")
        wid = s * NC + c
        for j in range(EB // 16):
            onesv[pl.ds(j * 16, 16)] = jnp.ones((16,), jnp.float32)
        for j in range(RPT // 16):
            stg[pl.ds(j * 16, 16)] = jnp.zeros((16,), jnp.float32)
        pltpu.sync_copy(stg, deg_sh.at[pl.ds(s * RPT, RPT)])
        plsc.subcore_barrier()

        nb = full + jnp.where(wid < rem, 1, 0)

        def body(j, carry):
            grow = wid + j * NW
            pltpu.sync_copy(col_hbm.at[grow], colv)
            pltpu.sync_copy(onesv, deg_sh.at[colv], add=True)
            return carry

        lax.fori_loop(0, nb, body, 0)
        plsc.subcore_barrier()
        pltpu.sync_copy(deg_sh.at[pl.ds(s * RPT, RPT)], stg)
        pltpu.sync_copy(stg, out_hbm.at[pl.ds(c * NPAD + s * RPT, RPT)])

    return deg_kernel(col2d)


def _sc_spmm(q, row2d, col2d):
    er = row2d.shape[0]
    full, rem = divmod(er, NW)

    @functools.partial(
        pl.kernel,
        mesh=_vsmesh(),
        out_type=jax.ShapeDtypeStruct((NC * NPAD, D), jnp.float32),
        scratch_types=[
            pltpu.VMEM((EB,), jnp.int32),
            pltpu.VMEM((EB,), jnp.int32),
            pltpu.VMEM((EB, D), jnp.float32),
            pltpu.VMEM_SHARED((NPAD, D), jnp.float32),
            pltpu.SemaphoreType.DMA,
        ],
    )
    def spmm_kernel(q_hbm, row_hbm, col_hbm, out_hbm, rowv, colv, gbuf, acc_sh, sem):
        c = lax.axis_index("c")
        s = lax.axis_index("s")
        wid = s * NC + c

        def zrow(r, carry):
            for j in range(D // 16):
                gbuf[r, pl.ds(j * 16, 16)] = jnp.zeros((16,), jnp.float32)
            return carry

        lax.fori_loop(0, EB, zrow, 0)
        for k in range(RPT // EB):
            pltpu.sync_copy(gbuf, acc_sh.at[pl.ds(s * RPT + k * EB, EB)])
        plsc.subcore_barrier()

        nb = full + jnp.where(wid < rem, 1, 0)

        def body(j, carry):
            grow = wid + j * NW
            pltpu.sync_copy(row_hbm.at[grow], rowv)
            pltpu.sync_copy(col_hbm.at[grow], colv)
            pltpu.async_copy(q_hbm.at[rowv], gbuf, sem).wait()
            pltpu.sync_copy(gbuf, acc_sh.at[colv], add=True)
            return carry

        lax.fori_loop(0, nb, body, 0)
        plsc.subcore_barrier()
        for k in range(RPT // EB):
            pltpu.sync_copy(acc_sh.at[pl.ds(s * RPT + k * EB, EB)], gbuf)
            pltpu.sync_copy(
                gbuf, out_hbm.at[pl.ds(c * NPAD + s * RPT + k * EB, EB)]
            )

    return spmm_kernel(q, row2d, col2d)


def _scale(deg_blk):
    return lax.rsqrt(1.0 + deg_blk[:, 0:1] + deg_blk[:, 1:2])


def _tc1(x, fc_W, fc_b, W1, degT):
    def body(x_ref, fcw_ref, fcb_ref, w1_ref, deg_ref, q_ref):
        h0 = jnp.dot(x_ref[...], fcw_ref[...], preferred_element_type=jnp.float32)
        h0 = h0 + fcb_ref[...]
        p1 = jnp.dot(h0, w1_ref[...], preferred_element_type=jnp.float32)
        q_ref[...] = p1 * _scale(deg_ref[...])

    return pl.pallas_call(
        body,
        grid=(N // R,),
        in_specs=[
            pl.BlockSpec((R, D), lambda i: (i, 0)),
            pl.BlockSpec((D, D), lambda i: (0, 0)),
            pl.BlockSpec((1, D), lambda i: (0, 0)),
            pl.BlockSpec((D, D), lambda i: (0, 0)),
            pl.BlockSpec((R, NC), lambda i: (i, 0)),
        ],
        out_specs=pl.BlockSpec((R, D), lambda i: (i, 0)),
        out_shape=jax.ShapeDtypeStruct((N, D), jnp.float32),
    )(x, fc_W, fc_b, W1, degT)


def _tc2(agg, q1, degT, b1, W2):
    def body(agg_ref, q1_ref, deg_ref, b1_ref, w2_ref, q2_ref):
        sc = _scale(deg_ref[...])
        a = agg_ref[0] + agg_ref[1] + q1_ref[...]
        h = jnp.maximum(sc * a + b1_ref[...], 0.0)
        p2 = jnp.dot(h, w2_ref[...], preferred_element_type=jnp.float32)
        q2_ref[...] = p2 * sc

    return pl.pallas_call(
        body,
        grid=(N // R,),
        in_specs=[
            pl.BlockSpec((NC, R, D), lambda i: (0, i, 0)),
            pl.BlockSpec((R, D), lambda i: (i, 0)),
            pl.BlockSpec((R, NC), lambda i: (i, 0)),
            pl.BlockSpec((1, D), lambda i: (0, 0)),
            pl.BlockSpec((D, D), lambda i: (0, 0)),
        ],
        out_specs=pl.BlockSpec((R, D), lambda i: (i, 0)),
        out_shape=jax.ShapeDtypeStruct((N, D), jnp.float32),
    )(agg, q1, degT, b1, W2)


def _tc3(agg, q2, degT, b2):
    def body(agg_ref, q2_ref, deg_ref, b2_ref, o_ref):
        sc = _scale(deg_ref[...])
        a = agg_ref[0] + agg_ref[1] + q2_ref[...]
        o_ref[...] = sc * a + b2_ref[...]

    return pl.pallas_call(
        body,
        grid=(N // R,),
        in_specs=[
            pl.BlockSpec((NC, R, D), lambda i: (0, i, 0)),
            pl.BlockSpec((R, D), lambda i: (i, 0)),
            pl.BlockSpec((R, NC), lambda i: (i, 0)),
            pl.BlockSpec((1, D), lambda i: (0, 0)),
        ],
        out_specs=pl.BlockSpec((R, D), lambda i: (i, 0)),
        out_shape=jax.ShapeDtypeStruct((N, D), jnp.float32),
    )(agg, q2, degT, b2)


def kernel(x, edge_index, fc_W, fc_b, W1, b1, W2, b2):
    ei = edge_index.astype(jnp.int32)
    e = ei.shape[1]
    er = e // EB
    row2d = ei[0].reshape(er, EB)
    col2d = ei[1].reshape(er, EB)

    deg_flat = _sc_degree(col2d)
    degT = deg_flat.reshape(NC, NPAD).T

    q1 = _tc1(x, fc_W, fc_b.reshape(1, D), W1, degT)
    agg1 = _sc_spmm(q1, row2d, col2d).reshape(NC, NPAD, D)
    q2 = _tc2(agg1, q1, degT, b1.reshape(1, D), W2)
    agg2 = _sc_spmm(q2, row2d, col2d).reshape(NC, NPAD, D)
    return _tc3(agg2, q2, degT, b2.reshape(1, D))
